# baseline (speedup 1.0000x reference)
import dataclasses
import functools

import jax
import jax.numpy as jnp
from jax import lax
from jax.experimental import pallas as pl
from jax.experimental.pallas import tpu as pltpu
from jax.experimental.pallas import tpu_sc as plsc

_N = 10000
_E = 320000
_P = 100000

_NC = 2
_NS = 16
_NT = _NC * _NS
_W = 4


def _sc_compiler_params():
    cp = pltpu.CompilerParams()
    if "needs_layout_passes" in pltpu.CompilerParams.__dataclass_fields__:
        cp = dataclasses.replace(cp, needs_layout_passes=False)
    return cp


def _mmt_body(x_ref, w_ref, o_ref):
    res = jnp.dot(x_ref[...], w_ref[...], preferred_element_type=jnp.float32)
    o_ref[...] = res.T


def _tc_matmul_t(x, w, blk=10000):
    n, k = x.shape
    h = w.shape[1]
    return pl.pallas_call(
        _mmt_body,
        grid=(n // blk,),
        in_specs=[pl.BlockSpec((blk, k), lambda i: (i, 0)),
                  pl.BlockSpec((k, h), lambda i: (0, 0))],
        out_specs=pl.BlockSpec((h, blk), lambda i: (0, i)),
        out_shape=jax.ShapeDtypeStruct((h, n), jnp.float32),
    )(x, w)


def _comb_body(transpose_out, p_ref, b_ref, w_ref, o_ref):
    p = p_ref[0]
    for g in range(1, p_ref.shape[0]):
        p = p + p_ref[g]
    hcur = jnp.maximum(p.T + b_ref[...], 0.0)
    res = jnp.dot(hcur, w_ref[...], preferred_element_type=jnp.float32)
    o_ref[...] = res.T if transpose_out else res


def _tc_combine_relu_matmul(parts, b, w, transpose_out, blk=10000):
    g, d, n = parts.shape
    h = w.shape[1]
    if transpose_out:
        out_specs = pl.BlockSpec((h, blk), lambda i: (0, i))
        out_shape = jax.ShapeDtypeStruct((h, n), jnp.float32)
    else:
        out_specs = pl.BlockSpec((blk, h), lambda i: (i, 0))
        out_shape = jax.ShapeDtypeStruct((n, h), jnp.float32)
    return pl.pallas_call(
        functools.partial(_comb_body, transpose_out),
        grid=(n // blk,),
        in_specs=[pl.BlockSpec((g, d, blk), lambda i: (0, 0, i)),
                  pl.BlockSpec((1, d), lambda i: (0, 0)),
                  pl.BlockSpec((d, h), lambda i: (0, 0))],
        out_specs=out_specs,
        out_shape=out_shape,
    )(parts, b, w)


def _decode_proj_body(p_ref, b_ref, w_ref, bd_ref, oa_ref, ob_ref):
    p = p_ref[0]
    for g in range(1, p_ref.shape[0]):
        p = p + p_ref[g]
    hcur = jnp.maximum(p.T + b_ref[...], 0.0)
    res = jnp.dot(hcur, w_ref[...], preferred_element_type=jnp.float32)
    res = res + bd_ref[...]
    oa_ref[...] = res[:, :1]
    ob_ref[...] = res[:, 1:]


def _tc_decode_proj(parts, b, wcat, bd_vec, blk=10000):
    g, d, n = parts.shape
    return pl.pallas_call(
        _decode_proj_body,
        grid=(n // blk,),
        in_specs=[pl.BlockSpec((g, d, blk), lambda i: (0, 0, i)),
                  pl.BlockSpec((1, d), lambda i: (0, 0)),
                  pl.BlockSpec((d, 2), lambda i: (0, 0)),
                  pl.BlockSpec((1, 2), lambda i: (0, 0))],
        out_specs=[pl.BlockSpec((blk, 1), lambda i: (i, 0)),
                   pl.BlockSpec((blk, 1), lambda i: (i, 0))],
        out_shape=[jax.ShapeDtypeStruct((n, 1), jnp.float32),
                   jax.ShapeDtypeStruct((n, 1), jnp.float32)],
    )(parts, b, wcat, bd_vec)


def _sc_segment_sum(h2t_flat, src, dst, vals, d):
    n = _N
    ns = d // _W
    ng = _NT // ns
    ept = _E // ng
    K = 80
    nch = ept // K
    nz = _W * n

    mesh = plsc.VectorSubcoreMesh(core_axis_name="c", subcore_axis_name="s")

    @functools.partial(
        pl.kernel,
        out_type=jax.ShapeDtypeStruct((ng * d * n,), jnp.float32),
        mesh=mesh,
        compiler_params=_sc_compiler_params(),
        scratch_types=[
            pltpu.VMEM((K,), jnp.int32),
            pltpu.VMEM((K,), jnp.int32),
            pltpu.VMEM((K,), jnp.float32),
            pltpu.VMEM((nz,), jnp.float32),
            pltpu.VMEM((nz,), jnp.float32),
            pltpu.SemaphoreType.DMA,
        ],
    )
    def seg_kernel(h2t_hbm, src_hbm, dst_hbm, vals_hbm, out_hbm,
                   src_v, dst_v, vals_v, tab_v, acc_v, sem):
        cid = lax.axis_index("c")
        sid = lax.axis_index("s")
        fs = sid % ns
        grp = (sid // ns) * _NC + cid

        pltpu.sync_copy(h2t_hbm.at[pl.ds(fs * nz, nz)], tab_v)

        @pl.loop(0, nz, step=16)
        def _zero(i):
            acc_v[pl.ds(i, 16)] = jnp.zeros((16,), jnp.float32)

        base = grp * ept

        @pl.loop(0, nch)
        def _chunk(c):
            eb = base + c * K
            pltpu.sync_copy(src_hbm.at[pl.ds(eb, K)], src_v)
            pltpu.sync_copy(dst_hbm.at[pl.ds(eb, K)], dst_v)
            pltpu.sync_copy(vals_hbm.at[pl.ds(eb, K)], vals_v)

            @pl.loop(0, K, step=16)
            def _step(e0):
                sl = pl.ds(e0, 16)
                s16 = src_v[sl]
                d16 = dst_v[sl]
                vv = vals_v[sl]
                for k in range(_W):
                    g = plsc.load_gather(tab_v, [s16 + (k * n)])
                    plsc.addupdate_scatter(acc_v, [d16 + (k * n)], g * vv)

        pltpu.sync_copy(acc_v, out_hbm.at[pl.ds(grp * (d * n) + fs * nz, nz)])

    return seg_kernel(h2t_flat, src, dst, vals)


def _sc_decode(a, b, rows, cols):
    n = a.shape[0]
    KP = 400
    nch = _P // KP
    jmax = (nch + _NT - 1) // _NT

    mesh = plsc.VectorSubcoreMesh(core_axis_name="c", subcore_axis_name="s")

    @functools.partial(
        pl.kernel,
        out_type=jax.ShapeDtypeStruct((_P,), jnp.float32),
        mesh=mesh,
        compiler_params=_sc_compiler_params(),
        scratch_types=[
            pltpu.VMEM((n,), jnp.float32),
            pltpu.VMEM((n,), jnp.float32),
            pltpu.VMEM((KP,), jnp.int32),
            pltpu.VMEM((KP,), jnp.int32),
            pltpu.VMEM((KP,), jnp.float32),
            pltpu.SemaphoreType.DMA,
        ],
    )
    def dec_kernel(a_hbm, b_hbm, r_hbm, c_hbm, out_hbm,
                   a_v, b_v, r_v, c_v, o_v, sem):
        cid = lax.axis_index("c")
        sid = lax.axis_index("s")
        tid = sid * _NC + cid
        pltpu.sync_copy(a_hbm, a_v)
        pltpu.sync_copy(b_hbm, b_v)

        @pl.loop(0, jmax)
        def _j(j):
            c = tid + _NT * j

            @pl.when(c < nch)
            def _():
                pb = c * KP
                pltpu.sync_copy(r_hbm.at[pl.ds(pb, KP)], r_v)
                pltpu.sync_copy(c_hbm.at[pl.ds(pb, KP)], c_v)
                for kk in range(0, KP, 16):
                    sl = pl.ds(kk, 16)
                    va = plsc.load_gather(a_v, [r_v[sl]])
                    vb = plsc.load_gather(b_v, [c_v[sl]])
                    o_v[sl] = va + vb
                pltpu.sync_copy(o_v, out_hbm.at[pl.ds(pb, KP)])

    return dec_kernel(a, b, rows, cols)


def kernel(node_features, support_indices, support_values, row_indices,
           col_indices, W0, b0, W1, b1, Wd, bd):
    src = support_indices[0]
    dst = support_indices[1]
    h0 = W0.shape[1]
    h1 = W1.shape[1]

    wcat = jnp.concatenate([Wd[:h1], Wd[h1:]], axis=1)
    bd_vec = jnp.stack([bd[0], jnp.zeros((), jnp.float32)])[None, :]

    h2at = _tc_matmul_t(node_features, W0)
    parts0 = _sc_segment_sum(h2at.reshape(-1), src, dst,
                             support_values, h0)
    h2bt = _tc_combine_relu_matmul(parts0.reshape(-1, h0, _N),
                                   b0[None, :], W1,
                                   transpose_out=True)
    parts1 = _sc_segment_sum(h2bt.reshape(-1), src, dst,
                             support_values, h1)
    av, bv = _tc_decode_proj(parts1.reshape(-1, h1, _N),
                             b1[None, :], wcat, bd_vec)
    out = _sc_decode(av[:, 0], bv[:, 0], row_indices, col_indices)
    return out[:, None]


# speedup vs baseline: 6.7710x; 6.7710x over previous
import dataclasses
import functools

import jax
import jax.numpy as jnp
from jax import lax
from jax.experimental import pallas as pl
from jax.experimental.pallas import tpu as pltpu
from jax.experimental.pallas import tpu_sc as plsc

_N = 10000
_E = 320000
_P = 100000

_NC = 2
_NS = 16
_NT = _NC * _NS
_W = 4


def _sc_compiler_params():
    cp = pltpu.CompilerParams()
    if "needs_layout_passes" in pltpu.CompilerParams.__dataclass_fields__:
        cp = dataclasses.replace(cp, needs_layout_passes=False)
    return cp


def _mmt_body(x_ref, w_ref, o_ref):
    res = jnp.dot(x_ref[...], w_ref[...], preferred_element_type=jnp.float32)
    o_ref[...] = res.T


def _tc_matmul_t(x, w, blk=10000):
    n, k = x.shape
    h = w.shape[1]
    return pl.pallas_call(
        _mmt_body,
        grid=(n // blk,),
        in_specs=[pl.BlockSpec((blk, k), lambda i: (i, 0)),
                  pl.BlockSpec((k, h), lambda i: (0, 0))],
        out_specs=pl.BlockSpec((h, blk), lambda i: (0, i)),
        out_shape=jax.ShapeDtypeStruct((h, n), jnp.float32),
    )(x, w)


def _comb_body(transpose_out, p_ref, b_ref, w_ref, o_ref):
    p = p_ref[0]
    for g in range(1, p_ref.shape[0]):
        p = p + p_ref[g]
    hcur = jnp.maximum(p.T + b_ref[...], 0.0)
    res = jnp.dot(hcur, w_ref[...], preferred_element_type=jnp.float32)
    o_ref[...] = res.T if transpose_out else res


def _tc_combine_relu_matmul(parts, b, w, transpose_out, blk=10000):
    g, d, n = parts.shape
    h = w.shape[1]
    if transpose_out:
        out_specs = pl.BlockSpec((h, blk), lambda i: (0, i))
        out_shape = jax.ShapeDtypeStruct((h, n), jnp.float32)
    else:
        out_specs = pl.BlockSpec((blk, h), lambda i: (i, 0))
        out_shape = jax.ShapeDtypeStruct((n, h), jnp.float32)
    return pl.pallas_call(
        functools.partial(_comb_body, transpose_out),
        grid=(n // blk,),
        in_specs=[pl.BlockSpec((g, d, blk), lambda i: (0, 0, i)),
                  pl.BlockSpec((1, d), lambda i: (0, 0)),
                  pl.BlockSpec((d, h), lambda i: (0, 0))],
        out_specs=out_specs,
        out_shape=out_shape,
    )(parts, b, w)


def _decode_proj_body(p_ref, b_ref, w_ref, bd_ref, oa_ref, ob_ref):
    p = p_ref[0]
    for g in range(1, p_ref.shape[0]):
        p = p + p_ref[g]
    hcur = jnp.maximum(p.T + b_ref[...], 0.0)
    res = jnp.dot(hcur, w_ref[...], preferred_element_type=jnp.float32)
    res = res + bd_ref[...]
    oa_ref[...] = res[:, :1]
    ob_ref[...] = res[:, 1:]


def _tc_decode_proj(parts, b, wcat, bd_vec, blk=10000):
    g, d, n = parts.shape
    return pl.pallas_call(
        _decode_proj_body,
        grid=(n // blk,),
        in_specs=[pl.BlockSpec((g, d, blk), lambda i: (0, 0, i)),
                  pl.BlockSpec((1, d), lambda i: (0, 0)),
                  pl.BlockSpec((d, 2), lambda i: (0, 0)),
                  pl.BlockSpec((1, 2), lambda i: (0, 0))],
        out_specs=[pl.BlockSpec((blk, 1), lambda i: (i, 0)),
                   pl.BlockSpec((blk, 1), lambda i: (i, 0))],
        out_shape=[jax.ShapeDtypeStruct((n, 1), jnp.float32),
                   jax.ShapeDtypeStruct((n, 1), jnp.float32)],
    )(parts, b, wcat, bd_vec)


def _sc_segment_sum(h2t_flat, src, dst, vals, d):
    n = _N
    ns = d // _W
    ng = _NT // ns
    ept = _E // ng
    K = 10000
    nch = ept // K
    nz = _W * n

    mesh = plsc.VectorSubcoreMesh(core_axis_name="c", subcore_axis_name="s")

    @functools.partial(
        pl.kernel,
        out_type=jax.ShapeDtypeStruct((ng * d * n,), jnp.float32),
        mesh=mesh,
        compiler_params=_sc_compiler_params(),
        scratch_types=[
            pltpu.VMEM((K,), jnp.int32),
            pltpu.VMEM((K,), jnp.int32),
            pltpu.VMEM((K,), jnp.float32),
            pltpu.VMEM((nz,), jnp.float32),
            pltpu.VMEM((nz,), jnp.float32),
            pltpu.SemaphoreType.DMA,
        ],
    )
    def seg_kernel(h2t_hbm, src_hbm, dst_hbm, vals_hbm, out_hbm,
                   src_v, dst_v, vals_v, tab_v, acc_v, sem):
        cid = lax.axis_index("c")
        sid = lax.axis_index("s")
        fs = sid % ns
        grp = (sid // ns) * _NC + cid

        pltpu.sync_copy(h2t_hbm.at[pl.ds(fs * nz, nz)], tab_v)

        @pl.loop(0, nz, step=16)
        def _zero(i):
            acc_v[pl.ds(i, 16)] = jnp.zeros((16,), jnp.float32)

        base = grp * ept

        @pl.loop(0, nch)
        def _chunk(c):
            eb = base + c * K
            pltpu.sync_copy(src_hbm.at[pl.ds(eb, K)], src_v)
            pltpu.sync_copy(dst_hbm.at[pl.ds(eb, K)], dst_v)
            pltpu.sync_copy(vals_hbm.at[pl.ds(eb, K)], vals_v)

            @pl.loop(0, K, step=16)
            def _step(e0):
                sl = pl.ds(e0, 16)
                s16 = src_v[sl]
                d16 = dst_v[sl]
                vv = vals_v[sl]
                for k in range(_W):
                    g = plsc.load_gather(tab_v, [s16 + (k * n)])
                    plsc.addupdate_scatter(acc_v, [d16 + (k * n)], g * vv)

        pltpu.sync_copy(acc_v, out_hbm.at[pl.ds(grp * (d * n) + fs * nz, nz)])

    return seg_kernel(h2t_flat, src, dst, vals)


def _sc_decode(a, b, rows, cols):
    n = a.shape[0]
    KP = 400
    nch = _P // KP
    jmax = (nch + _NT - 1) // _NT

    mesh = plsc.VectorSubcoreMesh(core_axis_name="c", subcore_axis_name="s")

    @functools.partial(
        pl.kernel,
        out_type=jax.ShapeDtypeStruct((_P,), jnp.float32),
        mesh=mesh,
        compiler_params=_sc_compiler_params(),
        scratch_types=[
            pltpu.VMEM((n,), jnp.float32),
            pltpu.VMEM((n,), jnp.float32),
            pltpu.VMEM((KP,), jnp.int32),
            pltpu.VMEM((KP,), jnp.int32),
            pltpu.VMEM((KP,), jnp.float32),
            pltpu.SemaphoreType.DMA,
        ],
    )
    def dec_kernel(a_hbm, b_hbm, r_hbm, c_hbm, out_hbm,
                   a_v, b_v, r_v, c_v, o_v, sem):
        cid = lax.axis_index("c")
        sid = lax.axis_index("s")
        tid = sid * _NC + cid
        pltpu.sync_copy(a_hbm, a_v)
        pltpu.sync_copy(b_hbm, b_v)

        @pl.loop(0, jmax)
        def _j(j):
            c = tid + _NT * j

            @pl.when(c < nch)
            def _():
                pb = c * KP
                pltpu.sync_copy(r_hbm.at[pl.ds(pb, KP)], r_v)
                pltpu.sync_copy(c_hbm.at[pl.ds(pb, KP)], c_v)
                for kk in range(0, KP, 16):
                    sl = pl.ds(kk, 16)
                    va = plsc.load_gather(a_v, [r_v[sl]])
                    vb = plsc.load_gather(b_v, [c_v[sl]])
                    o_v[sl] = va + vb
                pltpu.sync_copy(o_v, out_hbm.at[pl.ds(pb, KP)])

    return dec_kernel(a, b, rows, cols)


def kernel(node_features, support_indices, support_values, row_indices,
           col_indices, W0, b0, W1, b1, Wd, bd):
    src = support_indices[0]
    dst = support_indices[1]
    h0 = W0.shape[1]
    h1 = W1.shape[1]

    wcat = jnp.concatenate([Wd[:h1], Wd[h1:]], axis=1)
    bd_vec = jnp.stack([bd[0], jnp.zeros((), jnp.float32)])[None, :]

    h2at = _tc_matmul_t(node_features, W0)
    parts0 = _sc_segment_sum(h2at.reshape(-1), src, dst,
                             support_values, h0)
    h2bt = _tc_combine_relu_matmul(parts0.reshape(-1, h0, _N),
                                   b0[None, :], W1,
                                   transpose_out=True)
    parts1 = _sc_segment_sum(h2bt.reshape(-1), src, dst,
                             support_values, h1)
    av, bv = _tc_decode_proj(parts1.reshape(-1, h1, _N),
                             b1[None, :], wcat, bd_vec)
    out = _sc_decode(av[:, 0], bv[:, 0], row_indices, col_indices)
    return out[:, None]


# speedup vs baseline: 12.3911x; 1.8300x over previous
import dataclasses
import functools

import jax
import jax.numpy as jnp
from jax import lax
from jax.experimental import pallas as pl
from jax.experimental.pallas import tpu as pltpu
from jax.experimental.pallas import tpu_sc as plsc

_N = 10000
_E = 320000
_P = 100000

_NC = 2
_NS = 16
_NT = _NC * _NS
_W = 4


def _sc_compiler_params():
    cp = pltpu.CompilerParams()
    if "needs_layout_passes" in pltpu.CompilerParams.__dataclass_fields__:
        cp = dataclasses.replace(cp, needs_layout_passes=False)
    return cp


def _mmt_body(x_ref, w_ref, o_ref):
    res = jnp.dot(x_ref[...], w_ref[...], preferred_element_type=jnp.float32)
    o_ref[...] = res.T


def _tc_matmul_t(x, w, blk=10000):
    n, k = x.shape
    h = w.shape[1]
    return pl.pallas_call(
        _mmt_body,
        grid=(n // blk,),
        in_specs=[pl.BlockSpec((blk, k), lambda i: (i, 0)),
                  pl.BlockSpec((k, h), lambda i: (0, 0))],
        out_specs=pl.BlockSpec((h, blk), lambda i: (0, i)),
        out_shape=jax.ShapeDtypeStruct((h, n), jnp.float32),
    )(x, w)


def _comb_body(transpose_out, p_ref, b_ref, w_ref, o_ref):
    p = p_ref[0]
    for g in range(1, p_ref.shape[0]):
        p = p + p_ref[g]
    hcur = jnp.maximum(p.T + b_ref[...], 0.0)
    res = jnp.dot(hcur, w_ref[...], preferred_element_type=jnp.float32)
    o_ref[...] = res.T if transpose_out else res


def _tc_combine_relu_matmul(parts, b, w, transpose_out, blk=10000):
    g, d, n = parts.shape
    h = w.shape[1]
    if transpose_out:
        out_specs = pl.BlockSpec((h, blk), lambda i: (0, i))
        out_shape = jax.ShapeDtypeStruct((h, n), jnp.float32)
    else:
        out_specs = pl.BlockSpec((blk, h), lambda i: (i, 0))
        out_shape = jax.ShapeDtypeStruct((n, h), jnp.float32)
    return pl.pallas_call(
        functools.partial(_comb_body, transpose_out),
        grid=(n // blk,),
        in_specs=[pl.BlockSpec((g, d, blk), lambda i: (0, 0, i)),
                  pl.BlockSpec((1, d), lambda i: (0, 0)),
                  pl.BlockSpec((d, h), lambda i: (0, 0))],
        out_specs=out_specs,
        out_shape=out_shape,
    )(parts, b, w)


def _decode_proj_body(p_ref, b_ref, w_ref, bd_ref, oa_ref, ob_ref):
    p = p_ref[0]
    for g in range(1, p_ref.shape[0]):
        p = p + p_ref[g]
    hcur = jnp.maximum(p.T + b_ref[...], 0.0)
    res = jnp.dot(hcur, w_ref[...], preferred_element_type=jnp.float32)
    res = res + bd_ref[...]
    oa_ref[...] = res[:, :1]
    ob_ref[...] = res[:, 1:]


def _tc_decode_proj(parts, b, wcat, bd_vec, blk=10000):
    g, d, n = parts.shape
    return pl.pallas_call(
        _decode_proj_body,
        grid=(n // blk,),
        in_specs=[pl.BlockSpec((g, d, blk), lambda i: (0, 0, i)),
                  pl.BlockSpec((1, d), lambda i: (0, 0)),
                  pl.BlockSpec((d, 2), lambda i: (0, 0)),
                  pl.BlockSpec((1, 2), lambda i: (0, 0))],
        out_specs=[pl.BlockSpec((blk, 1), lambda i: (i, 0)),
                   pl.BlockSpec((blk, 1), lambda i: (i, 0))],
        out_shape=[jax.ShapeDtypeStruct((n, 1), jnp.float32),
                   jax.ShapeDtypeStruct((n, 1), jnp.float32)],
    )(parts, b, wcat, bd_vec)


def _sc_segment_sum(h2t_flat, src, dst, vals, d):
    n = _N
    ns = d // _W
    ng = _NT // ns
    ept = _E // ng
    K = 10000
    nch = ept // K
    nz = _W * n

    mesh = plsc.VectorSubcoreMesh(core_axis_name="c", subcore_axis_name="s")

    @functools.partial(
        pl.kernel,
        out_type=jax.ShapeDtypeStruct((ng * d * n,), jnp.float32),
        mesh=mesh,
        compiler_params=_sc_compiler_params(),
        scratch_types=[
            pltpu.VMEM((K,), jnp.int32),
            pltpu.VMEM((K,), jnp.int32),
            pltpu.VMEM((K,), jnp.float32),
            pltpu.VMEM((nz,), jnp.float32),
            pltpu.VMEM((nz,), jnp.float32),
            pltpu.SemaphoreType.DMA,
        ],
    )
    def seg_kernel(h2t_hbm, src_hbm, dst_hbm, vals_hbm, out_hbm,
                   src_v, dst_v, vals_v, tab_v, acc_v, sem):
        cid = lax.axis_index("c")
        sid = lax.axis_index("s")
        fs = sid % ns
        grp = (sid // ns) * _NC + cid

        pltpu.sync_copy(h2t_hbm.at[pl.ds(fs * nz, nz)], tab_v)

        @pl.loop(0, nz, step=16)
        def _zero(i):
            acc_v[pl.ds(i, 16)] = jnp.zeros((16,), jnp.float32)

        base = grp * ept

        @pl.loop(0, nch)
        def _chunk(c):
            eb = base + c * K
            pltpu.sync_copy(src_hbm.at[pl.ds(eb, K)], src_v)
            pltpu.sync_copy(dst_hbm.at[pl.ds(eb, K)], dst_v)
            pltpu.sync_copy(vals_hbm.at[pl.ds(eb, K)], vals_v)

            @plsc.parallel_loop(0, K, step=16, unroll=2)
            def _step(e0):
                sl = pl.ds(e0, 16)
                s16 = src_v[sl]
                d16 = dst_v[sl]
                vv = vals_v[sl]
                for k in range(_W):
                    g = plsc.load_gather(tab_v, [s16 + (k * n)])
                    plsc.addupdate_scatter(acc_v, [d16 + (k * n)], g * vv)

        pltpu.sync_copy(acc_v, out_hbm.at[pl.ds(grp * (d * n) + fs * nz, nz)])

    return seg_kernel(h2t_flat, src, dst, vals)


def _sc_decode(a, b, rows, cols):
    n = a.shape[0]
    KP = 400
    nch = _P // KP
    jmax = (nch + _NT - 1) // _NT

    mesh = plsc.VectorSubcoreMesh(core_axis_name="c", subcore_axis_name="s")

    @functools.partial(
        pl.kernel,
        out_type=jax.ShapeDtypeStruct((_P,), jnp.float32),
        mesh=mesh,
        compiler_params=_sc_compiler_params(),
        scratch_types=[
            pltpu.VMEM((n,), jnp.float32),
            pltpu.VMEM((n,), jnp.float32),
            pltpu.VMEM((KP,), jnp.int32),
            pltpu.VMEM((KP,), jnp.int32),
            pltpu.VMEM((KP,), jnp.float32),
            pltpu.SemaphoreType.DMA,
        ],
    )
    def dec_kernel(a_hbm, b_hbm, r_hbm, c_hbm, out_hbm,
                   a_v, b_v, r_v, c_v, o_v, sem):
        cid = lax.axis_index("c")
        sid = lax.axis_index("s")
        tid = sid * _NC + cid
        pltpu.sync_copy(a_hbm, a_v)
        pltpu.sync_copy(b_hbm, b_v)

        @pl.loop(0, jmax)
        def _j(j):
            c = tid + _NT * j

            @pl.when(c < nch)
            def _():
                pb = c * KP
                pltpu.sync_copy(r_hbm.at[pl.ds(pb, KP)], r_v)
                pltpu.sync_copy(c_hbm.at[pl.ds(pb, KP)], c_v)
                for kk in range(0, KP, 16):
                    sl = pl.ds(kk, 16)
                    va = plsc.load_gather(a_v, [r_v[sl]])
                    vb = plsc.load_gather(b_v, [c_v[sl]])
                    o_v[sl] = va + vb
                pltpu.sync_copy(o_v, out_hbm.at[pl.ds(pb, KP)])

    return dec_kernel(a, b, rows, cols)


def kernel(node_features, support_indices, support_values, row_indices,
           col_indices, W0, b0, W1, b1, Wd, bd):
    src = support_indices[0]
    dst = support_indices[1]
    h0 = W0.shape[1]
    h1 = W1.shape[1]

    wcat = jnp.concatenate([Wd[:h1], Wd[h1:]], axis=1)
    bd_vec = jnp.stack([bd[0], jnp.zeros((), jnp.float32)])[None, :]

    h2at = _tc_matmul_t(node_features, W0)
    parts0 = _sc_segment_sum(h2at.reshape(-1), src, dst,
                             support_values, h0)
    h2bt = _tc_combine_relu_matmul(parts0.reshape(-1, h0, _N),
                                   b0[None, :], W1,
                                   transpose_out=True)
    parts1 = _sc_segment_sum(h2bt.reshape(-1), src, dst,
                             support_values, h1)
    av, bv = _tc_decode_proj(parts1.reshape(-1, h1, _N),
                             b1[None, :], wcat, bd_vec)
    out = _sc_decode(av[:, 0], bv[:, 0], row_indices, col_indices)
    return out[:, None]
